# baseline (speedup 1.0000x reference)
import jax
import jax.numpy as jnp
from jax import lax
from jax.experimental import pallas as pl
from jax.experimental.pallas import tpu as pltpu
from jax.experimental.pallas import tpu_sc as plsc

N_NODES = 10000
N_EDGES = 640000
D = 128
DH = 64
EPS = 1e-16


N_TILES = 16
EDGES_PER_TILE = N_EDGES // N_TILES
CHUNK = 80
N_CHUNKS = EDGES_PER_TILE // CHUNK
ROWS_PER_TILE = 624
DRAIN = 48
N_DRAIN = ROWS_PER_TILE // DRAIN
TAIL_BASE = ROWS_PER_TILE * N_TILES
TAIL = N_NODES - TAIL_BASE


def _edge_body(x0_hbm, x1_hbm, attl_hbm, attr_hbm, src_hbm, dst_hbm, out_hbm,
               src_v, dst_v, srow, drow, obuf, atl_v, atr_v, zrow, acc_sh,
               sem_i, sem_g, sem_s):
    cid = lax.axis_index("c")
    sid = lax.axis_index("s")

    def run_core(x_hbm, c):
        pltpu.sync_copy(attl_hbm.at[pl.ds(c * DH, DH)], atl_v)
        pltpu.sync_copy(attr_hbm.at[pl.ds(c * DH, DH)], atr_v)

        def zfill(i, carry):
            for b in range(4):
                zrow[i, pl.ds(32 * b, 32)] = jnp.zeros((32,), jnp.bfloat16)
            return carry
        lax.fori_loop(0, DRAIN, zfill, 0)
        for k in range(N_DRAIN):
            pltpu.sync_copy(zrow, acc_sh.at[pl.ds(sid * ROWS_PER_TILE + k * DRAIN, DRAIN)])

        @pl.when(sid == N_TILES - 1)
        def _():
            pltpu.sync_copy(zrow.at[pl.ds(0, TAIL)], acc_sh.at[pl.ds(TAIL_BASE, TAIL)])
        plsc.subcore_barrier()

        base0 = sid * EDGES_PER_TILE

        def idx_start(k, b):
            base = base0 + k * CHUNK
            pltpu.async_copy(src_hbm.at[pl.ds(base, CHUNK)], src_v[b], sem_i[b])
            pltpu.async_copy(dst_hbm.at[pl.ds(base, CHUNK)], dst_v[b], sem_i[b])

        def idx_wait(k, b):
            base = base0 + k * CHUNK
            pltpu.make_async_copy(src_hbm.at[pl.ds(base, CHUNK)], src_v[b], sem_i[b]).wait()
            pltpu.make_async_copy(dst_hbm.at[pl.ds(base, CHUNK)], dst_v[b], sem_i[b]).wait()

        def gathers_start(db, ib):
            pltpu.async_copy(x_hbm.at[src_v[ib]], srow[db], sem_g[db])
            pltpu.async_copy(x_hbm.at[dst_v[ib]], drow[db], sem_g[db])

        def gathers_wait(db, ib):
            pltpu.make_async_copy(x_hbm.at[src_v[ib]], srow[db], sem_g[db]).wait()
            pltpu.make_async_copy(x_hbm.at[dst_v[ib]], drow[db], sem_g[db]).wait()

        def scatter_start(db, ib):
            pltpu.async_copy(obuf[db], acc_sh.at[dst_v[ib]], sem_s[db], add=True)

        def scatter_wait(db, ib):
            pltpu.make_async_copy(obuf[db], acc_sh.at[dst_v[ib]], sem_s[db]).wait()

        def compute(b):
            mask_hi = jnp.full((16,), -65536, jnp.int32)

            @plsc.parallel_loop(0, CHUNK, step=1, unroll=4)
            def _(e):
                for blk2 in range(2):
                    ws = srow[b][e, pl.ds(16 * blk2, 16)]
                    wd = drow[b][e, pl.ds(16 * blk2, 16)]
                    xs_pair = (plsc.bitcast(ws << 16, jnp.float32),
                               plsc.bitcast(ws & mask_hi, jnp.float32))
                    xd_pair = (plsc.bitcast(wd << 16, jnp.float32),
                               plsc.bitcast(wd & mask_hi, jnp.float32))
                    ts = []
                    ms = []
                    for h in range(2):
                        sl = pl.ds(32 * blk2 + 16 * h, 16)
                        xs = xs_pair[h]
                        xd = xd_pair[h]
                        z = atl_v[sl] * xs + atr_v[sl] * xd
                        z = jnp.minimum(jnp.maximum(z, z * 0.2), 80.0)
                        t = jnp.exp(z)
                        ts.append(t)
                        ms.append(xs * t)

                    def pack2(a, bb):
                        ia = plsc.bitcast(a, jnp.int32)
                        ib = plsc.bitcast(bb, jnp.int32)
                        w = lax.shift_right_logical(ia + 0x8000, 16) | ((ib + 0x8000) & mask_hi)
                        return plsc.bitcast(w, jnp.bfloat16)
                    obuf[b][e, pl.ds(32 * blk2, 32)] = pack2(ts[0], ts[1])
                    obuf[b][e, pl.ds(DH + 32 * blk2, 32)] = pack2(ms[0], ms[1])

        pltpu.sync_copy(src_hbm.at[pl.ds(base0, CHUNK)], src_v[0])
        pltpu.sync_copy(dst_hbm.at[pl.ds(base0, CHUNK)], dst_v[0])
        gathers_start(0, 0)
        idx_start(1, 1)

        def group_body(j, carry):
            for b in range(4):
                k = 4 * j + b
                db = b % 2
                ndb = 1 - db
                gathers_wait(db, b)

                @pl.when(k < N_CHUNKS - 1)
                def _():
                    idx_wait(k + 1, (b + 1) % 4)
                    gathers_start(ndb, (b + 1) % 4)

                @pl.when(k < N_CHUNKS - 2)
                def _():
                    idx_start(k + 2, (b + 2) % 4)
                compute(db)

                @pl.when(k > 0)
                def _():
                    scatter_wait(ndb, (b + 3) % 4)
                scatter_start(db, b)
            return carry
        lax.fori_loop(0, N_CHUNKS // 4, group_body, 0)
        scatter_wait(1, 3)

        plsc.subcore_barrier()
        r0 = sid * ROWS_PER_TILE
        pltpu.sync_copy(acc_sh.at[pl.ds(r0, ROWS_PER_TILE)],
                        out_hbm.at[c, pl.ds(r0, ROWS_PER_TILE)])

        @pl.when(sid == N_TILES - 1)
        def _():
            pltpu.sync_copy(acc_sh.at[pl.ds(TAIL_BASE, TAIL)],
                            out_hbm.at[c, pl.ds(TAIL_BASE, TAIL)])

    @pl.when(cid == 0)
    def _():
        run_core(x0_hbm, 0)

    @pl.when(cid == 1)
    def _():
        run_core(x1_hbm, 1)


_edge_pass = pl.kernel(
    _edge_body,
    out_type=jax.ShapeDtypeStruct((2, N_NODES, D), jnp.bfloat16),
    mesh=plsc.VectorSubcoreMesh(core_axis_name="c", subcore_axis_name="s"),
    compiler_params=pltpu.CompilerParams(use_tc_tiling_on_sc=False, needs_layout_passes=False),
    scratch_types=[
        (pltpu.VMEM((CHUNK,), jnp.int32),) * 4,
        (pltpu.VMEM((CHUNK,), jnp.int32),) * 4,
        (pltpu.VMEM((CHUNK, DH // 2), jnp.int32),) * 2,
        (pltpu.VMEM((CHUNK, DH // 2), jnp.int32),) * 2,
        (pltpu.VMEM((CHUNK, D), jnp.bfloat16),) * 2,
        pltpu.VMEM((DH,), jnp.float32),
        pltpu.VMEM((DH,), jnp.float32),
        pltpu.VMEM((DRAIN, D), jnp.bfloat16),
        pltpu.VMEM_SHARED((N_NODES, D), jnp.bfloat16),
        (pltpu.SemaphoreType.DMA,) * 4,
        (pltpu.SemaphoreType.DMA,) * 2,
        (pltpu.SemaphoreType.DMA,) * 2,
    ],
)


ROW_BLK = 1000
GRID = (N_NODES // ROW_BLK,)


def _pack_bf16(a, b):
    ia = lax.bitcast_convert_type(a, jnp.int32)
    ib = lax.bitcast_convert_type(b, jnp.int32)
    lo = lax.shift_right_logical(ia + 0x8000, 16)
    hi = (ib + 0x8000) & jnp.int32(-65536)
    return lo | hi


def _pack_half(xl_half):
    return jnp.concatenate(
        [_pack_bf16(xl_half[:, 0:16], xl_half[:, 16:32]),
         _pack_bf16(xl_half[:, 32:48], xl_half[:, 48:64])], axis=1)


def _xform1_body(h_ref, w_ref, b_ref, x0, x1):
    xl = lax.dot_general(h_ref[...], w_ref[...], (((1,), (1,)), ((), ())),
                         preferred_element_type=jnp.float32) + b_ref[...]
    x0[...] = _pack_half(xl[:, :DH])
    x1[...] = _pack_half(xl[:, DH:])


def _deint32(v):
    n = v.shape[0]
    return jnp.swapaxes(v.reshape(n, 16, 2), 1, 2).reshape(n, 32)


def _deint64(v):
    return jnp.concatenate([_deint32(v[:, :32]), _deint32(v[:, 32:])], axis=1)


def _norm_h(o):
    o = o.astype(jnp.float32)
    h0 = _deint64(o[0, :, DH:]) / (_deint64(o[0, :, :DH]) + EPS)
    h1 = _deint64(o[1, :, DH:]) / (_deint64(o[1, :, :DH]) + EPS)
    return jnp.maximum(jnp.concatenate([h0, h1], axis=1), 0.0)


def _xform2_body(o_ref, w_ref, b_ref, x0, x1):
    h = _norm_h(o_ref[...])
    xl = lax.dot_general(h, w_ref[...], (((1,), (1,)), ((), ())),
                         preferred_element_type=jnp.float32) + b_ref[...]
    x0[...] = _pack_half(xl[:, :DH])
    x1[...] = _pack_half(xl[:, DH:])


def _final_body(o_ref, pw1_ref, pb1_ref, pw2_ref, pb2_ref, out_ref):
    h = _norm_h(o_ref[...])
    z = lax.dot_general(h, pw1_ref[...], (((1,), (1,)), ((), ())),
                        preferred_element_type=jnp.float32) + pb1_ref[...]
    y = lax.dot_general(z, pw2_ref[...], (((1,), (1,)), ((), ())),
                        preferred_element_type=jnp.float32) + pb2_ref[...]
    t = y - jnp.max(y, axis=1, keepdims=True)
    out_ref[...] = t - jnp.log(jnp.sum(jnp.exp(t), axis=1, keepdims=True))


def _row_spec(cols):
    return pl.BlockSpec((ROW_BLK, cols), lambda i: (i, 0))


def _full_spec(shape):
    n = len(shape)
    return pl.BlockSpec(shape, lambda i: (0,) * n)


_table_out_shapes = (
    jax.ShapeDtypeStruct((N_NODES, DH // 2), jnp.int32),
    jax.ShapeDtypeStruct((N_NODES, DH // 2), jnp.int32),
)
_table_out_specs = (_row_spec(DH // 2), _row_spec(DH // 2))
_osp = pl.BlockSpec((2, ROW_BLK, D), lambda i: (0, i, 0))


def _xform1(x, W, b):
    return pl.pallas_call(
        _xform1_body, grid=GRID,
        in_specs=[_row_spec(D), _full_spec((D, D)), _full_spec((1, D))],
        out_specs=_table_out_specs, out_shape=_table_out_shapes,
    )(x, W, b)


def _xform2(o, W, b):
    return pl.pallas_call(
        _xform2_body, grid=GRID,
        in_specs=[_osp, _full_spec((D, D)), _full_spec((1, D))],
        out_specs=_table_out_specs, out_shape=_table_out_shapes,
    )(o, W, b)


def _final(o, pW1, pb1, pW2, pb2):
    return pl.pallas_call(
        _final_body, grid=GRID,
        in_specs=[_osp, _full_spec((DH, D)), _full_spec((1, DH)),
                  _full_spec((D, DH)), _full_spec((1, D))],
        out_specs=_row_spec(D),
        out_shape=jax.ShapeDtypeStruct((N_NODES, D), jnp.float32),
    )(o, pW1, pb1, pW2, pb2)


def kernel(x, edge_index, batch, W1, b1, attl1, attr1, W2, b2, attl2, attr2,
           pW1, pb1, pW2, pb2):
    src = edge_index[0].astype(jnp.int32)
    dst = edge_index[1].astype(jnp.int32)
    x0, x1 = _xform1(x, W1, b1.reshape(1, D))
    o1 = _edge_pass(x0, x1, attl1.reshape(D), attr1.reshape(D), src, dst)
    x0, x1 = _xform2(o1, W2, b2.reshape(1, D))
    o2 = _edge_pass(x0, x1, attl2.reshape(D), attr2.reshape(D), src, dst)
    return _final(o2, pW1, pb1.reshape(1, DH), pW2, pb2.reshape(1, D))


# speedup vs baseline: 1.8711x; 1.8711x over previous
import jax
import jax.numpy as jnp
from jax import lax
from jax.experimental import pallas as pl
from jax.experimental.pallas import tpu as pltpu
from jax.experimental.pallas import tpu_sc as plsc

N_NODES = 10000
N_EDGES = 640000
D = 128
DH = 64
EPS = 1e-16


N_TILES = 16
EDGES_PER_TILE = N_EDGES // N_TILES
CHUNK = 128
N_CHUNKS = 312
LEFT_BASE = N_TILES * N_CHUNKS * CHUNK
ROWS_PER_TILE = 624
DRAIN = 48
N_DRAIN = ROWS_PER_TILE // DRAIN
TAIL_BASE = ROWS_PER_TILE * N_TILES
TAIL = N_NODES - TAIL_BASE


def _edge_body(x0_hbm, x1_hbm, attl_hbm, attr_hbm, src_hbm, dst_hbm, out_hbm,
               src_v, dst_v, srow, drow, obuf, atl_v, atr_v, acc_sh,
               sem_i, sem_g, sem_s):
    cid = lax.axis_index("c")
    sid = lax.axis_index("s")

    def run_core(x_hbm, c):
        pltpu.sync_copy(attl_hbm.at[pl.ds(c * DH, DH)], atl_v)
        pltpu.sync_copy(attr_hbm.at[pl.ds(c * DH, DH)], atr_v)

        def zfill(i, carry):
            for b in range(8):
                obuf[0][i, pl.ds(16 * b, 16)] = jnp.zeros((16,), jnp.float32)
            return carry
        lax.fori_loop(0, CHUNK, zfill, 0)
        for k in range(4):
            pltpu.sync_copy(obuf[0], acc_sh.at[pl.ds(sid * ROWS_PER_TILE + k * CHUNK, CHUNK)])
        pltpu.sync_copy(obuf[0].at[pl.ds(0, ROWS_PER_TILE - 4 * CHUNK)],
                        acc_sh.at[pl.ds(sid * ROWS_PER_TILE + 4 * CHUNK, ROWS_PER_TILE - 4 * CHUNK)])

        @pl.when(sid == N_TILES - 1)
        def _():
            pltpu.sync_copy(obuf[0].at[pl.ds(0, TAIL)], acc_sh.at[pl.ds(TAIL_BASE, TAIL)])
        plsc.subcore_barrier()

        base0 = sid * (N_CHUNKS * CHUNK)

        def idx_start(k, b):
            base = base0 + k * CHUNK
            pltpu.async_copy(src_hbm.at[pl.ds(base, CHUNK)], src_v[b], sem_i[b])
            pltpu.async_copy(dst_hbm.at[pl.ds(base, CHUNK)], dst_v[b], sem_i[b])

        def idx_wait(k, b):
            base = base0 + k * CHUNK
            pltpu.make_async_copy(src_hbm.at[pl.ds(base, CHUNK)], src_v[b], sem_i[b]).wait()
            pltpu.make_async_copy(dst_hbm.at[pl.ds(base, CHUNK)], dst_v[b], sem_i[b]).wait()

        def gathers_start(db, ib):
            pltpu.async_copy(x_hbm.at[src_v[ib]], srow[db], sem_g[db])
            pltpu.async_copy(x_hbm.at[dst_v[ib]], drow[db], sem_g[db])

        def gathers_wait(db, ib):
            pltpu.make_async_copy(x_hbm.at[src_v[ib]], srow[db], sem_g[db]).wait()
            pltpu.make_async_copy(x_hbm.at[dst_v[ib]], drow[db], sem_g[db]).wait()

        def scatter_start(db, ib):
            pltpu.async_copy(obuf[db], acc_sh.at[dst_v[ib]], sem_s[db], add=True)

        def scatter_wait(db, ib):
            pltpu.make_async_copy(obuf[db], acc_sh.at[dst_v[ib]], sem_s[db]).wait()

        def compute(b):
            mask_hi = jnp.full((16,), -65536, jnp.int32)

            @plsc.parallel_loop(0, CHUNK, step=1, unroll=4)
            def _(e):
                for blk2 in range(2):
                    ws = srow[b][e, pl.ds(16 * blk2, 16)]
                    wd = drow[b][e, pl.ds(16 * blk2, 16)]
                    xs_pair = (plsc.bitcast(ws << 16, jnp.float32),
                               plsc.bitcast(ws & mask_hi, jnp.float32))
                    xd_pair = (plsc.bitcast(wd << 16, jnp.float32),
                               plsc.bitcast(wd & mask_hi, jnp.float32))
                    for h in range(2):
                        off = 32 * blk2 + 16 * h
                        sl = pl.ds(off, 16)
                        xs = xs_pair[h]
                        xd = xd_pair[h]
                        z = atl_v[sl] * xs + atr_v[sl] * xd
                        z = jnp.minimum(jnp.maximum(z, z * 0.2), 80.0)
                        t = jnp.exp(z)
                        obuf[b][e, sl] = t
                        obuf[b][e, pl.ds(DH + off, 16)] = xs * t

        pltpu.sync_copy(src_hbm.at[pl.ds(base0, CHUNK)], src_v[0])
        pltpu.sync_copy(dst_hbm.at[pl.ds(base0, CHUNK)], dst_v[0])
        gathers_start(0, 0)
        idx_start(1, 1)

        def group_body(j, carry):
            for b in range(4):
                k = 4 * j + b
                db = b % 2
                ndb = 1 - db
                gathers_wait(db, b)

                @pl.when(k < N_CHUNKS - 1)
                def _():
                    idx_wait(k + 1, (b + 1) % 4)
                    gathers_start(ndb, (b + 1) % 4)

                @pl.when(k < N_CHUNKS - 2)
                def _():
                    idx_start(k + 2, (b + 2) % 4)
                compute(db)

                @pl.when(k > 0)
                def _():
                    scatter_wait(ndb, (b + 3) % 4)
                scatter_start(db, b)
            return carry
        lax.fori_loop(0, N_CHUNKS // 4, group_body, 0)
        scatter_wait(1, 3)

        @pl.when(sid < 8)
        def _():
            lbase = LEFT_BASE + sid * CHUNK
            pltpu.sync_copy(src_hbm.at[pl.ds(lbase, CHUNK)], src_v[0])
            pltpu.sync_copy(dst_hbm.at[pl.ds(lbase, CHUNK)], dst_v[0])
            pltpu.async_copy(x_hbm.at[src_v[0]], srow[0], sem_g[0]).wait()
            pltpu.async_copy(x_hbm.at[dst_v[0]], drow[0], sem_g[0]).wait()
            compute(0)
            pltpu.sync_copy(obuf[0], acc_sh.at[dst_v[0]], add=True)

        plsc.subcore_barrier()
        r0 = sid * ROWS_PER_TILE
        pltpu.sync_copy(acc_sh.at[pl.ds(r0, ROWS_PER_TILE)],
                        out_hbm.at[c, pl.ds(r0, ROWS_PER_TILE)])

        @pl.when(sid == N_TILES - 1)
        def _():
            pltpu.sync_copy(acc_sh.at[pl.ds(TAIL_BASE, TAIL)],
                            out_hbm.at[c, pl.ds(TAIL_BASE, TAIL)])

    @pl.when(cid == 0)
    def _():
        run_core(x0_hbm, 0)

    @pl.when(cid == 1)
    def _():
        run_core(x1_hbm, 1)


_edge_pass = pl.kernel(
    _edge_body,
    out_type=jax.ShapeDtypeStruct((2, N_NODES, D), jnp.float32),
    mesh=plsc.VectorSubcoreMesh(core_axis_name="c", subcore_axis_name="s"),
    compiler_params=pltpu.CompilerParams(use_tc_tiling_on_sc=False, needs_layout_passes=False),
    scratch_types=[
        (pltpu.VMEM((CHUNK,), jnp.int32),) * 4,
        (pltpu.VMEM((CHUNK,), jnp.int32),) * 4,
        (pltpu.VMEM((CHUNK, DH // 2), jnp.int32),) * 2,
        (pltpu.VMEM((CHUNK, DH // 2), jnp.int32),) * 2,
        (pltpu.VMEM((CHUNK, D), jnp.float32),) * 2,
        pltpu.VMEM((DH,), jnp.float32),
        pltpu.VMEM((DH,), jnp.float32),
        pltpu.VMEM_SHARED((N_NODES, D), jnp.float32),
        (pltpu.SemaphoreType.DMA,) * 4,
        (pltpu.SemaphoreType.DMA,) * 2,
        (pltpu.SemaphoreType.DMA,) * 2,
    ],
)


ROW_BLK = 1000
GRID = (N_NODES // ROW_BLK,)


def _pack_bf16(a, b):
    ia = lax.bitcast_convert_type(a, jnp.int32)
    ib = lax.bitcast_convert_type(b, jnp.int32)
    lo = lax.shift_right_logical(ia + 0x8000, 16)
    hi = (ib + 0x8000) & jnp.int32(-65536)
    return lo | hi


def _pack_half(xl_half):
    return jnp.concatenate(
        [_pack_bf16(xl_half[:, 0:16], xl_half[:, 16:32]),
         _pack_bf16(xl_half[:, 32:48], xl_half[:, 48:64])], axis=1)


def _xform1_body(h_ref, w_ref, b_ref, x0, x1):
    xl = lax.dot_general(h_ref[...], w_ref[...], (((1,), (1,)), ((), ())),
                         preferred_element_type=jnp.float32) + b_ref[...]
    x0[...] = _pack_half(xl[:, :DH])
    x1[...] = _pack_half(xl[:, DH:])


def _norm_h(o):
    h0 = o[0, :, DH:] / (o[0, :, :DH] + EPS)
    h1 = o[1, :, DH:] / (o[1, :, :DH] + EPS)
    return jnp.maximum(jnp.concatenate([h0, h1], axis=1), 0.0)


def _xform2_body(o_ref, w_ref, b_ref, x0, x1):
    h = _norm_h(o_ref[...])
    xl = lax.dot_general(h, w_ref[...], (((1,), (1,)), ((), ())),
                         preferred_element_type=jnp.float32) + b_ref[...]
    x0[...] = _pack_half(xl[:, :DH])
    x1[...] = _pack_half(xl[:, DH:])


def _final_body(o_ref, pw1_ref, pb1_ref, pw2_ref, pb2_ref, out_ref):
    h = _norm_h(o_ref[...])
    z = lax.dot_general(h, pw1_ref[...], (((1,), (1,)), ((), ())),
                        preferred_element_type=jnp.float32) + pb1_ref[...]
    y = lax.dot_general(z, pw2_ref[...], (((1,), (1,)), ((), ())),
                        preferred_element_type=jnp.float32) + pb2_ref[...]
    t = y - jnp.max(y, axis=1, keepdims=True)
    out_ref[...] = t - jnp.log(jnp.sum(jnp.exp(t), axis=1, keepdims=True))


def _row_spec(cols):
    return pl.BlockSpec((ROW_BLK, cols), lambda i: (i, 0))


def _full_spec(shape):
    n = len(shape)
    return pl.BlockSpec(shape, lambda i: (0,) * n)


_table_out_shapes = (
    jax.ShapeDtypeStruct((N_NODES, DH // 2), jnp.int32),
    jax.ShapeDtypeStruct((N_NODES, DH // 2), jnp.int32),
)
_table_out_specs = (_row_spec(DH // 2), _row_spec(DH // 2))
_osp = pl.BlockSpec((2, ROW_BLK, D), lambda i: (0, i, 0))


def _xform1(x, W, b):
    return pl.pallas_call(
        _xform1_body, grid=GRID,
        in_specs=[_row_spec(D), _full_spec((D, D)), _full_spec((1, D))],
        out_specs=_table_out_specs, out_shape=_table_out_shapes,
    )(x, W, b)


def _xform2(o, W, b):
    return pl.pallas_call(
        _xform2_body, grid=GRID,
        in_specs=[_osp, _full_spec((D, D)), _full_spec((1, D))],
        out_specs=_table_out_specs, out_shape=_table_out_shapes,
    )(o, W, b)


def _final(o, pW1, pb1, pW2, pb2):
    return pl.pallas_call(
        _final_body, grid=GRID,
        in_specs=[_osp, _full_spec((DH, D)), _full_spec((1, DH)),
                  _full_spec((D, DH)), _full_spec((1, D))],
        out_specs=_row_spec(D),
        out_shape=jax.ShapeDtypeStruct((N_NODES, D), jnp.float32),
    )(o, pW1, pb1, pW2, pb2)


def kernel(x, edge_index, batch, W1, b1, attl1, attr1, W2, b2, attl2, attr2,
           pW1, pb1, pW2, pb2):
    src = edge_index[0].astype(jnp.int32)
    dst = edge_index[1].astype(jnp.int32)
    x0, x1 = _xform1(x, W1, b1.reshape(1, D))
    o1 = _edge_pass(x0, x1, attl1.reshape(D), attr1.reshape(D), src, dst)
    x0, x1 = _xform2(o1, W2, b2.reshape(1, D))
    o2 = _edge_pass(x0, x1, attl2.reshape(D), attr2.reshape(D), src, dst)
    return _final(o2, pW1, pb1.reshape(1, DH), pW2, pb2.reshape(1, D))


# speedup vs baseline: 1.8718x; 1.0004x over previous
import jax
import jax.numpy as jnp
from jax import lax
from jax.experimental import pallas as pl
from jax.experimental.pallas import tpu as pltpu
from jax.experimental.pallas import tpu_sc as plsc

N_NODES = 10000
N_EDGES = 640000
D = 128
DH = 64
EPS = 1e-16


N_TILES = 16
CHUNK = 128
N_CHUNKS = 312
LEFT_BASE = N_TILES * N_CHUNKS * CHUNK
ROWS_PER_TILE = 624
TAIL_BASE = ROWS_PER_TILE * N_TILES
TAIL = N_NODES - TAIL_BASE


def _edge_body(x0_hbm, x1_hbm, attl_hbm, attr_hbm, src_hbm, dst_hbm, out_hbm,
               src_v, dst_v, srow, drow, obuf, atl_v, atr_v, acc_sh,
               sem_i, sem_g, sem_s):
    cid = lax.axis_index("c")
    sid = lax.axis_index("s")

    def run_core(x_hbm, c):
        pltpu.sync_copy(attl_hbm.at[pl.ds(c * DH, DH)], atl_v)
        pltpu.sync_copy(attr_hbm.at[pl.ds(c * DH, DH)], atr_v)

        def zfill(i, carry):
            for b in range(8):
                obuf[0][i, pl.ds(16 * b, 16)] = jnp.zeros((16,), jnp.float32)
            return carry
        lax.fori_loop(0, CHUNK, zfill, 0)
        for k in range(4):
            pltpu.sync_copy(obuf[0], acc_sh.at[pl.ds(sid * ROWS_PER_TILE + k * CHUNK, CHUNK)])
        pltpu.sync_copy(obuf[0].at[pl.ds(0, ROWS_PER_TILE - 4 * CHUNK)],
                        acc_sh.at[pl.ds(sid * ROWS_PER_TILE + 4 * CHUNK, ROWS_PER_TILE - 4 * CHUNK)])

        @pl.when(sid == N_TILES - 1)
        def _():
            pltpu.sync_copy(obuf[0].at[pl.ds(0, TAIL)], acc_sh.at[pl.ds(TAIL_BASE, TAIL)])
        plsc.subcore_barrier()

        base0 = sid * (N_CHUNKS * CHUNK)

        def idx_start(k, b):
            base = base0 + k * CHUNK
            pltpu.async_copy(src_hbm.at[pl.ds(base, CHUNK)], src_v[b], sem_i[b])
            pltpu.async_copy(dst_hbm.at[pl.ds(base, CHUNK)], dst_v[b], sem_i[b])

        def idx_wait(k, b):
            base = base0 + k * CHUNK
            pltpu.make_async_copy(src_hbm.at[pl.ds(base, CHUNK)], src_v[b], sem_i[b]).wait()
            pltpu.make_async_copy(dst_hbm.at[pl.ds(base, CHUNK)], dst_v[b], sem_i[b]).wait()

        def gathers_start(db, ib):
            pltpu.async_copy(x_hbm.at[src_v[ib]], srow[db], sem_g[db])
            pltpu.async_copy(x_hbm.at[dst_v[ib]], drow[db], sem_g[db])

        def gathers_wait(db, ib):
            pltpu.make_async_copy(x_hbm.at[src_v[ib]], srow[db], sem_g[db]).wait()
            pltpu.make_async_copy(x_hbm.at[dst_v[ib]], drow[db], sem_g[db]).wait()

        def scatter_start(db, ib):
            pltpu.async_copy(obuf[db], acc_sh.at[dst_v[ib]], sem_s[db], add=True)

        def scatter_wait(db, ib):
            pltpu.make_async_copy(obuf[db], acc_sh.at[dst_v[ib]], sem_s[db]).wait()

        def compute(b):
            mask_hi = jnp.full((16,), -65536, jnp.int32)

            @plsc.parallel_loop(0, CHUNK, step=1, unroll=4)
            def _(e):
                for blk2 in range(2):
                    ws = srow[b][e, pl.ds(16 * blk2, 16)]
                    wd = drow[b][e, pl.ds(16 * blk2, 16)]
                    xs_pair = (plsc.bitcast(ws << 16, jnp.float32),
                               plsc.bitcast(ws & mask_hi, jnp.float32))
                    xd_pair = (plsc.bitcast(wd << 16, jnp.float32),
                               plsc.bitcast(wd & mask_hi, jnp.float32))
                    for h in range(2):
                        off = 32 * blk2 + 16 * h
                        sl = pl.ds(off, 16)
                        xs = xs_pair[h]
                        xd = xd_pair[h]
                        z = atl_v[sl] * xs + atr_v[sl] * xd
                        z = jnp.minimum(jnp.maximum(z, z * 0.2), 80.0)
                        t = jnp.exp(z)
                        obuf[b][e, sl] = t
                        obuf[b][e, pl.ds(DH + off, 16)] = xs * t

        pltpu.sync_copy(src_hbm.at[pl.ds(base0, CHUNK)], src_v[0])
        pltpu.sync_copy(dst_hbm.at[pl.ds(base0, CHUNK)], dst_v[0])
        gathers_start(0, 0)
        idx_start(1, 1)

        def group_body(j, carry):
            for b in range(4):
                k = 4 * j + b
                db = b % 2
                ndb = 1 - db
                gathers_wait(db, b)

                @pl.when(k < N_CHUNKS - 1)
                def _():
                    idx_wait(k + 1, (b + 1) % 4)
                    gathers_start(ndb, (b + 1) % 4)

                @pl.when(k < N_CHUNKS - 2)
                def _():
                    idx_start(k + 2, (b + 2) % 4)
                compute(db)

                @pl.when(k > 0)
                def _():
                    scatter_wait(ndb, (b + 3) % 4)
                scatter_start(db, b)
            return carry
        lax.fori_loop(0, N_CHUNKS // 4, group_body, 0)
        scatter_wait(1, 3)

        @pl.when(sid < 8)
        def _():
            lbase = LEFT_BASE + sid * CHUNK
            pltpu.sync_copy(src_hbm.at[pl.ds(lbase, CHUNK)], src_v[0])
            pltpu.sync_copy(dst_hbm.at[pl.ds(lbase, CHUNK)], dst_v[0])
            pltpu.async_copy(x_hbm.at[src_v[0]], srow[0], sem_g[0]).wait()
            pltpu.async_copy(x_hbm.at[dst_v[0]], drow[0], sem_g[0]).wait()
            compute(0)
            pltpu.sync_copy(obuf[0], acc_sh.at[dst_v[0]], add=True)

        plsc.subcore_barrier()
        r0 = sid * ROWS_PER_TILE
        pltpu.sync_copy(acc_sh.at[pl.ds(r0, ROWS_PER_TILE)],
                        out_hbm.at[c, pl.ds(r0, ROWS_PER_TILE)])

        @pl.when(sid == N_TILES - 1)
        def _():
            pltpu.sync_copy(acc_sh.at[pl.ds(TAIL_BASE, TAIL)],
                            out_hbm.at[c, pl.ds(TAIL_BASE, TAIL)])

    @pl.when(cid == 0)
    def _():
        run_core(x0_hbm, 0)

    @pl.when(cid == 1)
    def _():
        run_core(x1_hbm, 1)


_edge_pass = pl.kernel(
    _edge_body,
    out_type=jax.ShapeDtypeStruct((2, N_NODES, D), jnp.float32),
    mesh=plsc.VectorSubcoreMesh(core_axis_name="c", subcore_axis_name="s"),
    compiler_params=pltpu.CompilerParams(use_tc_tiling_on_sc=False, needs_layout_passes=False),
    scratch_types=[
        (pltpu.VMEM((CHUNK,), jnp.int32),) * 4,
        (pltpu.VMEM((CHUNK,), jnp.int32),) * 4,
        (pltpu.VMEM((CHUNK, DH // 2), jnp.int32),) * 2,
        (pltpu.VMEM((CHUNK, DH // 2), jnp.int32),) * 2,
        (pltpu.VMEM((CHUNK, D), jnp.float32),) * 2,
        pltpu.VMEM((DH,), jnp.float32),
        pltpu.VMEM((DH,), jnp.float32),
        pltpu.VMEM_SHARED((N_NODES, D), jnp.float32),
        (pltpu.SemaphoreType.DMA,) * 4,
        (pltpu.SemaphoreType.DMA,) * 2,
        (pltpu.SemaphoreType.DMA,) * 2,
    ],
)


ROW_BLK = 1000
GRID = (N_NODES // ROW_BLK,)


def _pack_bf16(a, b):
    ia = lax.bitcast_convert_type(a, jnp.int32)
    ib = lax.bitcast_convert_type(b, jnp.int32)
    lo = lax.shift_right_logical(ia + 0x8000, 16)
    hi = (ib + 0x8000) & jnp.int32(-65536)
    return lo | hi


def _pack_half(xl_half):
    return jnp.concatenate(
        [_pack_bf16(xl_half[:, 0:16], xl_half[:, 16:32]),
         _pack_bf16(xl_half[:, 32:48], xl_half[:, 48:64])], axis=1)


def _xform1_body(h_ref, w_ref, b_ref, x0, x1):
    xl = lax.dot_general(h_ref[...], w_ref[...], (((1,), (1,)), ((), ())),
                         preferred_element_type=jnp.float32) + b_ref[...]
    x0[...] = _pack_half(xl[:, :DH])
    x1[...] = _pack_half(xl[:, DH:])


def _norm_h(o):
    h0 = o[0, :, DH:] / (o[0, :, :DH] + EPS)
    h1 = o[1, :, DH:] / (o[1, :, :DH] + EPS)
    return jnp.maximum(jnp.concatenate([h0, h1], axis=1), 0.0)


def _xform2_body(o_ref, w_ref, b_ref, x0, x1):
    h = _norm_h(o_ref[...])
    xl = lax.dot_general(h, w_ref[...], (((1,), (1,)), ((), ())),
                         preferred_element_type=jnp.float32) + b_ref[...]
    x0[...] = _pack_half(xl[:, :DH])
    x1[...] = _pack_half(xl[:, DH:])


def _final_body(o_ref, pw1_ref, pb1_ref, pw2_ref, pb2_ref, out_ref):
    h = _norm_h(o_ref[...])
    z = lax.dot_general(h, pw1_ref[...], (((1,), (1,)), ((), ())),
                        preferred_element_type=jnp.float32) + pb1_ref[...]
    y = lax.dot_general(z, pw2_ref[...], (((1,), (1,)), ((), ())),
                        preferred_element_type=jnp.float32) + pb2_ref[...]
    t = y - jnp.max(y, axis=1, keepdims=True)
    out_ref[...] = t - jnp.log(jnp.sum(jnp.exp(t), axis=1, keepdims=True))


def _row_spec(cols):
    return pl.BlockSpec((ROW_BLK, cols), lambda i: (i, 0))


def _full_spec(shape):
    n = len(shape)
    return pl.BlockSpec(shape, lambda i: (0,) * n)


_table_out_shapes = (
    jax.ShapeDtypeStruct((N_NODES, DH // 2), jnp.int32),
    jax.ShapeDtypeStruct((N_NODES, DH // 2), jnp.int32),
)
_table_out_specs = (_row_spec(DH // 2), _row_spec(DH // 2))
_osp = pl.BlockSpec((2, ROW_BLK, D), lambda i: (0, i, 0))


def _xform1(x, W, b):
    return pl.pallas_call(
        _xform1_body, grid=GRID,
        in_specs=[_row_spec(D), _full_spec((D, D)), _full_spec((1, D))],
        out_specs=_table_out_specs, out_shape=_table_out_shapes,
    )(x, W, b)


def _xform2(o, W, b):
    return pl.pallas_call(
        _xform2_body, grid=GRID,
        in_specs=[_osp, _full_spec((D, D)), _full_spec((1, D))],
        out_specs=_table_out_specs, out_shape=_table_out_shapes,
    )(o, W, b)


def _final(o, pW1, pb1, pW2, pb2):
    return pl.pallas_call(
        _final_body, grid=GRID,
        in_specs=[_osp, _full_spec((DH, D)), _full_spec((1, DH)),
                  _full_spec((D, DH)), _full_spec((1, D))],
        out_specs=_row_spec(D),
        out_shape=jax.ShapeDtypeStruct((N_NODES, D), jnp.float32),
    )(o, pW1, pb1, pW2, pb2)


def kernel(x, edge_index, batch, W1, b1, attl1, attr1, W2, b2, attl2, attr2,
           pW1, pb1, pW2, pb2):
    src = edge_index[0].astype(jnp.int32)
    dst = edge_index[1].astype(jnp.int32)
    x0, x1 = _xform1(x, W1, b1.reshape(1, D))
    o1 = _edge_pass(x0, x1, attl1.reshape(D), attr1.reshape(D), src, dst)
    x0, x1 = _xform2(o1, W2, b2.reshape(1, D))
    o2 = _edge_pass(x0, x1, attl2.reshape(D), attr2.reshape(D), src, dst)
    return _final(o2, pW1, pb1.reshape(1, DH), pW2, pb2.reshape(1, D))
